# Initial kernel scaffold; baseline (speedup 1.0000x reference)
import jax
import jax.numpy as jnp
from jax.experimental import pallas as pl


def kernel(input_h, relation_embed, edges, W1, b1, L1, W2, b2, L2):
    raise NotImplementedError("write your pallas kernel here")


# speedup vs baseline: 3.4596x; 3.4596x over previous
import math

import jax
import jax.numpy as jnp
from jax import lax
from jax.experimental import pallas as pl
from jax.experimental.pallas import tpu as pltpu
from jax.experimental.pallas import tpu_sc as plsc

NC = 2
NS = 16
NW = NC * NS
CH = 128


def _sc_segsum(n_chunks, np_rows, d, with_deg, table, gidx, sidx):
    rows_per_tile = np_rows // NS
    copies_per_tile = rows_per_tile // CH
    njo = n_chunks // 2

    out_type = [jax.ShapeDtypeStruct((NC, np_rows, d), jnp.float32)]
    if with_deg:
        out_type.append(jax.ShapeDtypeStruct((NC, np_rows), jnp.float32))

    scratch = [
        pltpu.VMEM((n_chunks, CH), jnp.int32),
        pltpu.VMEM((CH,), jnp.int32),
        pltpu.VMEM((CH,), jnp.int32),
        pltpu.VMEM((CH, d), jnp.float32),
        pltpu.VMEM((CH, d), jnp.float32),
        pltpu.VMEM_SHARED((np_rows, d), jnp.float32),
        pltpu.SemaphoreType.DMA,
        pltpu.SemaphoreType.DMA,
        pltpu.SemaphoreType.DMA,
        pltpu.SemaphoreType.DMA,
        pltpu.SemaphoreType.DMA,
        pltpu.SemaphoreType.DMA,
    ]
    if with_deg:
        scratch += [
            pltpu.VMEM((CH,), jnp.float32),
            pltpu.VMEM((rows_per_tile,), jnp.float32),
            pltpu.VMEM_SHARED((np_rows,), jnp.float32),
            pltpu.SemaphoreType.DMA,
            pltpu.SemaphoreType.DMA,
        ]

    def body(table_hbm, gidx_hbm, sidx_hbm, *refs):
        if with_deg:
            (out_hbm, outd_hbm, sidx_v, gib0, gib1, rows0, rows1, acc,
             gs0, gs1, ss0, ss1, gis0, gis1,
             ones_v, zdeg_v, dacc, ds0, ds1) = refs
        else:
            (out_hbm, sidx_v, gib0, gib1, rows0, rows1, acc,
             gs0, gs1, ss0, ss1, gis0, gis1) = refs

        cid = lax.axis_index("c")
        sid = lax.axis_index("s")
        wid = sid * NC + cid
        base = sid * rows_per_tile

        def zrow(i, _):
            for j in range(d // 16):
                rows0[i, pl.ds(j * 16, 16)] = jnp.zeros((16,), jnp.float32)
            return 0
        lax.fori_loop(0, CH, zrow, 0)
        for k in range(copies_per_tile):
            pltpu.sync_copy(rows0, acc.at[pl.ds(base + k * CH, CH)])
        if with_deg:
            def zdeg(i, _):
                zdeg_v[pl.ds(i * 16, 16)] = jnp.zeros((16,), jnp.float32)
                return 0
            lax.fori_loop(0, rows_per_tile // 16, zdeg, 0)
            for j in range(CH // 16):
                ones_v[pl.ds(j * 16, 16)] = jnp.ones((16,), jnp.float32)
            pltpu.sync_copy(zdeg_v, dacc.at[pl.ds(base, rows_per_tile)])

        pltpu.sync_copy(sidx_hbm.at[wid], sidx_v)
        pltpu.sync_copy(gidx_hbm.at[wid, 0], gib0)
        pltpu.sync_copy(gidx_hbm.at[wid, 1], gib1)

        plsc.subcore_barrier()

        def start_gather(gib, rows, sem):
            return pltpu.async_copy(table_hbm.at[gib], rows, sem)

        def wait_gather(gib, rows, sem):
            pltpu.make_async_copy(table_hbm.at[gib], rows, sem).wait()

        def start_scatter(rows, j, sem):
            return pltpu.async_copy(rows, acc.at[sidx_v.at[j]], sem, add=True)

        def start_deg(j, sem):
            return pltpu.async_copy(ones_v, dacc.at[sidx_v.at[j]], sem, add=True)

        start_gather(gib0, rows0, gs0)
        start_gather(gib1, rows1, gs1)

        def step(jo, _):
            j0 = 2 * jo
            j1 = j0 + 1
            wait_gather(gib0, rows0, gs0)
            gl0 = pltpu.async_copy(gidx_hbm.at[wid, j0 + 2], gib0, gis0)
            s0 = start_scatter(rows0, j0, ss0)
            if with_deg:
                d0 = start_deg(j0, ds0)
            wait_gather(gib1, rows1, gs1)
            gl1 = pltpu.async_copy(gidx_hbm.at[wid, j1 + 2], gib1, gis1)
            s1 = start_scatter(rows1, j1, ss1)
            if with_deg:
                d1 = start_deg(j1, ds1)
            s0.wait()
            gl0.wait()
            start_gather(gib0, rows0, gs0)
            s1.wait()
            gl1.wait()
            start_gather(gib1, rows1, gs1)
            if with_deg:
                d0.wait()
                d1.wait()
            return 0
        lax.fori_loop(0, njo - 1, step, 0)

        jl0 = n_chunks - 2
        jl1 = n_chunks - 1
        wait_gather(gib0, rows0, gs0)
        s0 = start_scatter(rows0, jl0, ss0)
        wait_gather(gib1, rows1, gs1)
        s1 = start_scatter(rows1, jl1, ss1)
        if with_deg:
            d0 = start_deg(jl0, ds0)
            d1 = start_deg(jl1, ds1)
            d0.wait()
            d1.wait()
        s0.wait()
        s1.wait()

        plsc.subcore_barrier()

        for k in range(copies_per_tile):
            pltpu.sync_copy(acc.at[pl.ds(base + k * CH, CH)],
                            out_hbm.at[cid, pl.ds(base + k * CH, CH)])
        if with_deg:
            pltpu.sync_copy(dacc.at[pl.ds(base, rows_per_tile)],
                            outd_hbm.at[cid, pl.ds(base, rows_per_tile)])

    mesh = plsc.VectorSubcoreMesh(core_axis_name="c", subcore_axis_name="s",
                                  num_cores=NC, num_subcores=NS)
    run = pl.kernel(body, out_type=out_type, mesh=mesh, scratch_types=scratch)
    res = run(table, gidx, sidx)
    return res if with_deg else res[0]


def _tc_layer(n, d, ph, pr, degm, h, W, b8, L):
    bn = 1000
    grid = n // bn

    def body(ph_ref, pr_ref, degm_ref, h_ref, W_ref, b_ref, L_ref, o_ref):
        agg = (ph_ref[0] + ph_ref[1] + pr_ref[0] + pr_ref[1]) / degm_ref[...]
        o = jnp.dot(agg, W_ref[...], preferred_element_type=jnp.float32)
        o = o + jnp.dot(h_ref[...], L_ref[...], preferred_element_type=jnp.float32)
        o = o + b_ref[0:1, :]
        o_ref[...] = jnp.maximum(o, 0.0)

    return pl.pallas_call(
        body,
        grid=(grid,),
        in_specs=[
            pl.BlockSpec((NC, bn, d), lambda i: (0, i, 0)),
            pl.BlockSpec((NC, bn, d), lambda i: (0, i, 0)),
            pl.BlockSpec((bn, d), lambda i: (i, 0)),
            pl.BlockSpec((bn, d), lambda i: (i, 0)),
            pl.BlockSpec((d, d), lambda i: (0, 0)),
            pl.BlockSpec((8, d), lambda i: (0, 0)),
            pl.BlockSpec((d, d), lambda i: (0, 0)),
        ],
        out_specs=pl.BlockSpec((bn, d), lambda i: (i, 0)),
        out_shape=jax.ShapeDtypeStruct((n, d), jnp.float32),
    )(ph, pr, degm, h, W, b8, L)


def kernel(input_h, relation_embed, edges, W1, b1, L1, W2, b2, L2):
    n, d = input_h.shape
    e = edges.shape[0]

    per_chunkset = NW * CH
    n_chunks = math.ceil(e / per_chunkset)
    if n_chunks % 2:
        n_chunks += 1
    ep = n_chunks * per_chunkset
    np_rows = NS * CH * math.ceil((n + 1) / (NS * CH))

    pad = ep - e
    src = jnp.concatenate([edges[:, 0], jnp.zeros((pad,), jnp.int32)])
    rel = jnp.concatenate([edges[:, 1], jnp.zeros((pad,), jnp.int32)])
    dst = jnp.concatenate([edges[:, 2], jnp.full((pad,), n, jnp.int32)])
    src3 = src.reshape(NW, n_chunks, CH)
    rel3 = rel.reshape(NW, n_chunks, CH)
    dst3 = dst.reshape(NW, n_chunks, CH)

    pr, pdeg = _sc_segsum(n_chunks, np_rows, d, True, relation_embed, rel3, dst3)
    ph1 = _sc_segsum(n_chunks, np_rows, d, False, input_h, src3, dst3)

    deg = pdeg[0, :n] + pdeg[1, :n]
    degm = jnp.broadcast_to(jnp.maximum(deg, 1.0)[:, None], (n, d))
    b1p = jnp.broadcast_to(b1[None, :], (8, d))
    b2p = jnp.broadcast_to(b2[None, :], (8, d))

    out1 = _tc_layer(n, d, ph1, pr, degm, input_h, W1, b1p, L1)
    ph2 = _sc_segsum(n_chunks, np_rows, d, False, out1, src3, dst3)
    out2 = _tc_layer(n, d, ph2, pr, degm, out1, W2, b2p, L2)
    return out2
